# Initial kernel scaffold; baseline (speedup 1.0000x reference)
import jax
import jax.numpy as jnp
from jax.experimental import pallas as pl


def kernel(x, edge_index, edge_attr, batch, Wn, We, bm, Ws, Wa, bu, W1, b1, W2, b2):
    raise NotImplementedError("write your pallas kernel here")


# speedup vs baseline: 4.3890x; 4.3890x over previous
import functools

import jax
import jax.numpy as jnp
from jax import lax
from jax.experimental import pallas as pl
from jax.experimental.pallas import tpu as pltpu
from jax.experimental.pallas import tpu_sc as plsc

N = 10000
E = 320000
D = 128
ED = 16
H = 128
L = 3
G = 64
OUT = 128

NUM_CORES = 2
NUM_SUBCORES = 16
NW = NUM_CORES * NUM_SUBCORES
EW = E // NW
EB = 80
NBLK = EW // EB
ROW_SLAB = 624
ROW_TAIL_START = ROW_SLAB * NUM_SUBCORES
ROW_TAIL = N - ROW_TAIL_START


def _dense_pre_body(h_ref, wn_ref, ws_ref, p_ref, hs_ref):
    h = h_ref[...]
    p_ref[...] = jnp.dot(h, wn_ref[...], preferred_element_type=jnp.float32)
    hs_ref[...] = jnp.dot(h, ws_ref[...], preferred_element_type=jnp.float32)


def _q_body(ea_ref, we_ref, bm_ref, q_ref):
    q_ref[...] = (
        jnp.dot(ea_ref[...], we_ref[...], preferred_element_type=jnp.float32)
        + bm_ref[...]
    )


def _post_body(hs_ref, agg_ref, wa_ref, bu_ref, out_ref):
    agg = agg_ref[0] + agg_ref[1]
    out_ref[...] = jnp.maximum(
        jnp.dot(agg, wa_ref[...], preferred_element_type=jnp.float32)
        + hs_ref[...] + bu_ref[...],
        0.0,
    )


def _readout_body(h_ref, b_ref, w1_ref, b1_ref, w2_ref, b2_ref, out_ref):
    gi = lax.broadcasted_iota(jnp.int32, (G, N), 0)
    onehot = (gi == b_ref[...]).astype(jnp.float32)
    cnt = jnp.sum(onehot, axis=1, keepdims=True)
    pooled = jnp.dot(onehot, h_ref[...], preferred_element_type=jnp.float32)
    pooled = pooled / jnp.maximum(cnt, 1.0)
    t = jnp.maximum(
        jnp.dot(pooled, w1_ref[...], preferred_element_type=jnp.float32)
        + b1_ref[...],
        0.0,
    )
    out_ref[...] = (
        jnp.dot(t, w2_ref[...], preferred_element_type=jnp.float32) + b2_ref[...]
    )


_dense_pre = pl.pallas_call(
    _dense_pre_body,
    out_shape=[
        jax.ShapeDtypeStruct((N, H), jnp.float32),
        jax.ShapeDtypeStruct((N, H), jnp.float32),
    ],
)

_QB = 8000
_q_call = pl.pallas_call(
    _q_body,
    grid=(E // _QB,),
    in_specs=[
        pl.BlockSpec((_QB, ED), lambda i: (i, 0)),
        pl.BlockSpec((ED, H), lambda i: (0, 0)),
        pl.BlockSpec((1, H), lambda i: (0, 0)),
    ],
    out_specs=pl.BlockSpec((_QB, H), lambda i: (i, 0)),
    out_shape=jax.ShapeDtypeStruct((E, H), jnp.float32),
)

_post_call = pl.pallas_call(
    _post_body,
    out_shape=jax.ShapeDtypeStruct((N, H), jnp.float32),
)

_readout_call = pl.pallas_call(
    _readout_body,
    out_shape=jax.ShapeDtypeStruct((G, OUT), jnp.float32),
)


def _edge_body(p_hbm, q_hbm, src_hbm, dst_hbm, zero_hbm, out_hbm,
               idx_s, idx_d, rs, rd, qb, agg_sh, sem):
    c = lax.axis_index("c")
    s = lax.axis_index("s")
    wid = s * NUM_CORES + c

    r0 = s * ROW_SLAB
    pltpu.sync_copy(zero_hbm.at[pl.ds(r0, ROW_SLAB)],
                    agg_sh.at[pl.ds(r0, ROW_SLAB)])

    @pl.when(s == 0)
    def _zero_tail():
        pltpu.sync_copy(zero_hbm.at[pl.ds(ROW_TAIL_START, ROW_TAIL)],
                        agg_sh.at[pl.ds(ROW_TAIL_START, ROW_TAIL)])

    plsc.subcore_barrier()

    base0 = wid * EW

    def blk(i, carry):
        base = base0 + i * EB
        pltpu.sync_copy(src_hbm.at[pl.ds(base, EB)], idx_s)
        pltpu.sync_copy(dst_hbm.at[pl.ds(base, EB)], idx_d)
        cp1 = pltpu.async_copy(p_hbm.at[idx_s], rs, sem)
        cp2 = pltpu.async_copy(p_hbm.at[idx_d], rd, sem)
        cp3 = pltpu.async_copy(q_hbm.at[pl.ds(base, EB)], qb, sem)
        cp1.wait()
        cp2.wait()
        cp3.wait()

        def row(r, carry2):
            for j in range(H // 16):
                sl = pl.ds(j * 16, 16)
                v = rs[r, sl] + rd[r, sl] + qb[r, sl]
                rs[r, sl] = jnp.maximum(v, 0.0)
            return carry2

        lax.fori_loop(0, EB, row, 0)
        pltpu.sync_copy(rs, agg_sh.at[idx_s], add=True)
        pltpu.sync_copy(rs, agg_sh.at[idx_d], add=True)
        return carry

    lax.fori_loop(0, NBLK, blk, 0)
    plsc.subcore_barrier()
    pltpu.sync_copy(agg_sh.at[pl.ds(r0, ROW_SLAB)],
                    out_hbm.at[c, pl.ds(r0, ROW_SLAB)])

    @pl.when(s == 0)
    def _write_tail():
        pltpu.sync_copy(agg_sh.at[pl.ds(ROW_TAIL_START, ROW_TAIL)],
                        out_hbm.at[c, pl.ds(ROW_TAIL_START, ROW_TAIL)])


_edge_call = pl.kernel(
    _edge_body,
    out_type=jax.ShapeDtypeStruct((NUM_CORES, N, H), jnp.float32),
    mesh=plsc.VectorSubcoreMesh(
        core_axis_name="c", subcore_axis_name="s",
        num_cores=NUM_CORES, num_subcores=NUM_SUBCORES,
    ),
    scratch_types=[
        pltpu.VMEM((EB,), jnp.int32),
        pltpu.VMEM((EB,), jnp.int32),
        pltpu.VMEM((EB, H), jnp.float32),
        pltpu.VMEM((EB, H), jnp.float32),
        pltpu.VMEM((EB, H), jnp.float32),
        pltpu.VMEM_SHARED((N, H), jnp.float32),
        pltpu.SemaphoreType.DMA,
    ],
)


def kernel(x, edge_index, edge_attr, batch, Wn, We, bm, Ws, Wa, bu, W1, b1, W2, b2):
    src = edge_index[0]
    dst = edge_index[1]
    zeros = jnp.zeros((N, H), jnp.float32)
    bm2 = bm.reshape(L, 1, H)
    bu2 = bu.reshape(L, 1, H)
    batch2 = batch.reshape(1, N)

    h = x
    for l in range(L):
        p, hs = _dense_pre(h, Wn[l], Ws[l])
        q = _q_call(edge_attr, We[l], bm2[l])
        agg2 = _edge_call(p, q, src, dst, zeros)
        h = _post_call(hs, agg2, Wa[l], bu2[l])

    return _readout_call(h, batch2, W1, b1.reshape(1, H), W2, b2.reshape(1, OUT))


# speedup vs baseline: 6.0375x; 1.3756x over previous
import functools

import jax
import jax.numpy as jnp
from jax import lax
from jax.experimental import pallas as pl
from jax.experimental.pallas import tpu as pltpu
from jax.experimental.pallas import tpu_sc as plsc

N = 10000
E = 320000
D = 128
ED = 16
H = 128
L = 3
G = 64
OUT = 128

NUM_CORES = 2
NUM_SUBCORES = 16
NW = NUM_CORES * NUM_SUBCORES
EW = E // NW
EB = 40
NBLK = EW // EB
ROW_SLAB = 624
ROW_TAIL_START = ROW_SLAB * NUM_SUBCORES
ROW_TAIL = N - ROW_TAIL_START


def _dense_pre_body(h_ref, wn_ref, ws_ref, p_ref, hs_ref):
    h = h_ref[...]
    p_ref[...] = jnp.dot(h, wn_ref[...], preferred_element_type=jnp.float32)
    hs_ref[...] = jnp.dot(h, ws_ref[...], preferred_element_type=jnp.float32)


def _q_body(ea_ref, we_ref, bm_ref, q_ref):
    q_ref[...] = (
        jnp.dot(ea_ref[...], we_ref[...], preferred_element_type=jnp.float32)
        + bm_ref[...]
    )


def _post_body(hs_ref, agg_ref, wa_ref, bu_ref, out_ref):
    agg = agg_ref[0] + agg_ref[1]
    out_ref[...] = jnp.maximum(
        jnp.dot(agg, wa_ref[...], preferred_element_type=jnp.float32)
        + hs_ref[...] + bu_ref[...],
        0.0,
    )


def _readout_body(h_ref, b_ref, w1_ref, b1_ref, w2_ref, b2_ref, out_ref):
    gi = lax.broadcasted_iota(jnp.int32, (G, N), 0)
    onehot = (gi == b_ref[...]).astype(jnp.float32)
    cnt = jnp.sum(onehot, axis=1, keepdims=True)
    pooled = jnp.dot(onehot, h_ref[...], preferred_element_type=jnp.float32)
    pooled = pooled / jnp.maximum(cnt, 1.0)
    t = jnp.maximum(
        jnp.dot(pooled, w1_ref[...], preferred_element_type=jnp.float32)
        + b1_ref[...],
        0.0,
    )
    out_ref[...] = (
        jnp.dot(t, w2_ref[...], preferred_element_type=jnp.float32) + b2_ref[...]
    )


_dense_pre = pl.pallas_call(
    _dense_pre_body,
    out_shape=[
        jax.ShapeDtypeStruct((N, H), jnp.float32),
        jax.ShapeDtypeStruct((N, H), jnp.float32),
    ],
)

_QB = 8000
_q_call = pl.pallas_call(
    _q_body,
    grid=(E // _QB,),
    in_specs=[
        pl.BlockSpec((_QB, ED), lambda i: (i, 0)),
        pl.BlockSpec((ED, H), lambda i: (0, 0)),
        pl.BlockSpec((1, H), lambda i: (0, 0)),
    ],
    out_specs=pl.BlockSpec((_QB, H), lambda i: (i, 0)),
    out_shape=jax.ShapeDtypeStruct((E, H), jnp.float32),
)

_post_call = pl.pallas_call(
    _post_body,
    out_shape=jax.ShapeDtypeStruct((N, H), jnp.float32),
)

_readout_call = pl.pallas_call(
    _readout_body,
    out_shape=jax.ShapeDtypeStruct((G, OUT), jnp.float32),
)


def _edge_body(p_hbm, q_hbm, src_hbm, dst_hbm, zero_hbm, out_hbm,
               idx_s0, idx_d0, idx_s1, idx_d1,
               rs0, rd0, qb0, mb0, rs1, rd1, qb1, mb1,
               agg_sh, sem0, sem1, semi0, semi1):
    c = lax.axis_index("c")
    s = lax.axis_index("s")
    wid = s * NUM_CORES + c

    r0 = s * ROW_SLAB
    pltpu.sync_copy(zero_hbm.at[pl.ds(r0, ROW_SLAB)],
                    agg_sh.at[pl.ds(r0, ROW_SLAB)])

    @pl.when(s == 0)
    def _zero_tail():
        pltpu.sync_copy(zero_hbm.at[pl.ds(ROW_TAIL_START, ROW_TAIL)],
                        agg_sh.at[pl.ds(ROW_TAIL_START, ROW_TAIL)])

    plsc.subcore_barrier()

    base0 = wid * EW
    sets = (
        (idx_s0, idx_d0, rs0, rd0, qb0, mb0, sem0, semi0),
        (idx_s1, idx_d1, rs1, rd1, qb1, mb1, sem1, semi1),
    )

    def issue_idx(i, st):
        base = base0 + i * EB
        pltpu.async_copy(src_hbm.at[pl.ds(base, EB)], st[0], st[7])
        pltpu.async_copy(dst_hbm.at[pl.ds(base, EB)], st[1], st[7])

    def wait_idx(st):
        pltpu.make_async_copy(src_hbm.at[pl.ds(0, EB)], st[0], st[7]).wait()
        pltpu.make_async_copy(dst_hbm.at[pl.ds(0, EB)], st[1], st[7]).wait()

    def issue_gathers(i, st):
        base = base0 + i * EB
        pltpu.async_copy(p_hbm.at[st[0]], st[2], st[6])
        pltpu.async_copy(p_hbm.at[st[1]], st[3], st[6])
        pltpu.async_copy(q_hbm.at[pl.ds(base, EB)], st[4], st[6])

    def wait_gathers(st):
        pltpu.make_async_copy(p_hbm.at[st[0]], st[2], st[6]).wait()
        pltpu.make_async_copy(p_hbm.at[st[1]], st[3], st[6]).wait()
        pltpu.make_async_copy(q_hbm.at[pl.ds(0, EB)], st[4], st[6]).wait()

    def process(i, st, do_issue, guard_idx):
        wait_gathers(st)
        nxt = sets[1] if st is sets[0] else sets[0]
        if do_issue:
            wait_idx(nxt)
            issue_gathers(i + 1, nxt)
        rs, rd, qb, mb = st[2], st[3], st[4], st[5]

        def row(r, carry2):
            for j in range(H // 16):
                sl = pl.ds(j * 16, 16)
                v = rs[r, sl] + rd[r, sl] + qb[r, sl]
                mb[r, sl] = jnp.maximum(v, 0.0)
            return carry2

        lax.fori_loop(0, EB, row, 0)
        pltpu.sync_copy(mb, agg_sh.at[st[0]], add=True)
        pltpu.sync_copy(mb, agg_sh.at[st[1]], add=True)
        if do_issue:
            if guard_idx:
                @pl.when(i < NBLK - 2)
                def _issue():
                    issue_idx(i + 2, st)
            else:
                issue_idx(i + 2, st)

    issue_idx(0, sets[0])
    wait_idx(sets[0])
    issue_gathers(0, sets[0])
    issue_idx(1, sets[1])

    def pair(k, carry):
        process(2 * k, sets[0], do_issue=True, guard_idx=True)
        process(2 * k + 1, sets[1], do_issue=True, guard_idx=True)
        return carry

    lax.fori_loop(0, NBLK // 2 - 1, pair, 0)
    process(NBLK - 2, sets[0], do_issue=True, guard_idx=True)
    process(NBLK - 1, sets[1], do_issue=False, guard_idx=False)

    plsc.subcore_barrier()
    pltpu.sync_copy(agg_sh.at[pl.ds(r0, ROW_SLAB)],
                    out_hbm.at[c, pl.ds(r0, ROW_SLAB)])

    @pl.when(s == 0)
    def _write_tail():
        pltpu.sync_copy(agg_sh.at[pl.ds(ROW_TAIL_START, ROW_TAIL)],
                        out_hbm.at[c, pl.ds(ROW_TAIL_START, ROW_TAIL)])


_edge_call = pl.kernel(
    _edge_body,
    out_type=jax.ShapeDtypeStruct((NUM_CORES, N, H), jnp.float32),
    mesh=plsc.VectorSubcoreMesh(
        core_axis_name="c", subcore_axis_name="s",
        num_cores=NUM_CORES, num_subcores=NUM_SUBCORES,
    ),
    scratch_types=(
        [pltpu.VMEM((EB,), jnp.int32)] * 4
        + [pltpu.VMEM((EB, H), jnp.float32)] * 8
        + [pltpu.VMEM_SHARED((N, H), jnp.float32)]
        + [pltpu.SemaphoreType.DMA] * 4
    ),
)


def kernel(x, edge_index, edge_attr, batch, Wn, We, bm, Ws, Wa, bu, W1, b1, W2, b2):
    src = edge_index[0]
    dst = edge_index[1]
    zeros = jnp.zeros((N, H), jnp.float32)
    bm2 = bm.reshape(L, 1, H)
    bu2 = bu.reshape(L, 1, H)
    batch2 = batch.reshape(1, N)

    h = x
    for l in range(L):
        p, hs = _dense_pre(h, Wn[l], Ws[l])
        q = _q_call(edge_attr, We[l], bm2[l])
        agg2 = _edge_call(p, q, src, dst, zeros)
        h = _post_call(hs, agg2, Wa[l], bu2[l])

    return _readout_call(h, batch2, W1, b1.reshape(1, H), W2, b2.reshape(1, OUT))
